# baseline (speedup 1.0000x reference)
import functools

import jax
import jax.numpy as jnp
from jax.experimental import pallas as pl
from jax.experimental.pallas import tpu as pltpu
from jax.experimental.pallas import tpu_sc as plsc

_BN = 512
_GATHER_W = 128
_WIDE = 128


def _dist_body(z_ref, cb_ref, idx_ref, loss_ref, cbw_ref, cba_ref, acc_ref,
               *, denom):
    i = pl.program_id(0)
    nsteps = pl.num_programs(0)
    d = cb_ref.shape[1]

    @pl.when(i == 0)
    def _():
        cb = cb_ref[...]
        cba_ref[...] = jnp.sum(cb * cb, axis=1)
        acc_ref[...] = jnp.zeros_like(acc_ref)
        cbw_ref[:, :d] = cb
        cbw_ref[:, d:] = jnp.zeros((cb.shape[0], _WIDE - d), jnp.float32)

    z = z_ref[...]
    bn = z.shape[0]
    zsq = jnp.sum(z * z, axis=1, keepdims=True)
    dot = jax.lax.dot_general(
        z * -2.0, cb_ref[...], (((1,), (1,)), ((), ())),
        preferred_element_type=jnp.float32)
    csq = cba_ref[...]
    k = dot.shape[1]
    nl = 128
    rt = 64
    lane_iota = jax.lax.broadcasted_iota(jnp.int32, (rt, nl), 1)
    for r in range(bn // rt):
        rows = slice(r * rt, (r + 1) * rt)
        run_min = jnp.full((rt, nl), jnp.inf, jnp.float32)
        run_j = jnp.zeros((rt, nl), jnp.int32)
        for j in range(k // nl):
            d2j = ((zsq[rows, :] + dot[rows, j * nl:(j + 1) * nl])
                   + csq[j * nl:(j + 1) * nl][None, :])
            mask = d2j < run_min
            run_min = jnp.minimum(run_min, d2j)
            run_j = jnp.where(mask, j, run_j)
        m = jnp.min(run_min, axis=1, keepdims=True)
        tie = run_min == m
        glob = run_j * nl + lane_iota
        idx_ref[rows] = jnp.min(jnp.where(tie, glob, k), axis=1)
        acc_ref[rows, :] += m

    @pl.when(i == nsteps - 1)
    def _():
        loss_ref[0, 0] = jnp.sum(acc_ref[...]) / denom


def _tc_dist_argmin(z, codebook):
    n, d = z.shape
    k = codebook.shape[0]
    nb = n // _BN
    idx, loss, cb_wide = pl.pallas_call(
        functools.partial(_dist_body, denom=float(n * d)),
        grid=(nb,),
        in_specs=[
            pl.BlockSpec((_BN, d), lambda i: (i, 0)),
            pl.BlockSpec((k, d), lambda i: (0, 0)),
        ],
        out_specs=[
            pl.BlockSpec((_BN,), lambda i: (i,)),
            pl.BlockSpec(memory_space=pltpu.SMEM),
            pl.BlockSpec((k, _WIDE), lambda i: (0, 0)),
        ],
        out_shape=[
            jax.ShapeDtypeStruct((n,), jnp.int32),
            jax.ShapeDtypeStruct((1, 1), jnp.float32),
            jax.ShapeDtypeStruct((k, _WIDE), jnp.float32),
        ],
        scratch_shapes=[
            pltpu.VMEM((k,), jnp.float32),
            pltpu.VMEM((_BN, 1), jnp.float32),
        ],
        compiler_params=pltpu.CompilerParams(
            dimension_semantics=("arbitrary",)),
    )(z, codebook)
    return idx, loss, cb_wide


def _sc_gather_narrow(cb_wide, indices, d_out):
    n = indices.shape[0]
    dw = cb_wide.shape[1]
    idx2 = indices.reshape(1, n)
    mesh = plsc.VectorSubcoreMesh(
        core_axis_name="core", subcore_axis_name="subcore")

    @functools.partial(
        pl.kernel,
        out_type=jax.ShapeDtypeStruct((n, d_out), cb_wide.dtype),
        mesh=mesh,
        scratch_types=[pltpu.VMEM((_GATHER_W, dw), cb_wide.dtype)])
    def gather_kernel(cb_hbm, i_hbm, o_hbm, g_scratch):
        def body(i_vmem, o_vmem):
            pltpu.sync_copy(cb_hbm.at[i_vmem.at[0]], g_scratch)
            o_vmem[...] = g_scratch[:, :d_out]

        pltpu.emit_pipeline(
            body,
            grid=(n // _GATHER_W,),
            in_specs=[pl.BlockSpec((1, _GATHER_W), index_map=lambda i: (0, i))],
            out_specs=[pl.BlockSpec((_GATHER_W, d_out),
                                    index_map=lambda i: (i, 0))],
            core_axis_name=("core", "subcore"),
            dimension_semantics=(pltpu.PARALLEL,),
        )(i_hbm, o_hbm)

    return gather_kernel(cb_wide, idx2)


def kernel(z, codebook):
    idx, loss, cb_wide = _tc_dist_argmin(z, codebook)
    quantized = _sc_gather_narrow(cb_wide, idx, codebook.shape[1])
    return (quantized, idx, loss[0, 0])


# speedup vs baseline: 1.1641x; 1.1641x over previous
import functools

import jax
import jax.numpy as jnp
from jax.experimental import pallas as pl
from jax.experimental.pallas import tpu as pltpu
from jax.experimental.pallas import tpu_sc as plsc

_BN = 512
_GATHER_W = 128
_WIDE = 128
_CHUNKS = 4


def _prep_body(cb_ref, cbw_ref, csq_ref):
    d = cb_ref.shape[1]
    cb = cb_ref[...]
    cbw_ref[:, :d] = cb
    cbw_ref[:, d:] = jnp.zeros((cb.shape[0], _WIDE - d), jnp.float32)
    csq_ref[...] = jnp.sum(cb * cb, axis=1)


def _prep(codebook):
    k, _ = codebook.shape
    return pl.pallas_call(
        _prep_body,
        out_shape=[
            jax.ShapeDtypeStruct((k, _WIDE), jnp.float32),
            jax.ShapeDtypeStruct((k,), jnp.float32),
        ],
    )(codebook)


def _dist_body(z_ref, cbw_ref, csq_ref, idx_ref, lsum_ref, acc_ref, zp_ref):
    i = pl.program_id(0)
    nsteps = pl.num_programs(0)
    d = z_ref.shape[1]
    bn = z_ref.shape[0]

    @pl.when(i == 0)
    def _():
        acc_ref[...] = jnp.zeros_like(acc_ref)
        zp_ref[:, d:] = jnp.zeros((bn, _WIDE - d), jnp.float32)

    z = z_ref[...]
    zsq = jnp.sum(z * z, axis=1, keepdims=True)
    zp_ref[:, :d] = z * -2.0
    dot = jax.lax.dot_general(
        zp_ref[...], cbw_ref[...], (((1,), (1,)), ((), ())),
        preferred_element_type=jnp.float32)
    k = dot.shape[1]
    csq = csq_ref[...]
    nl = 128
    rt = 64
    lane_iota = jax.lax.broadcasted_iota(jnp.int32, (rt, nl), 1)
    for r in range(bn // rt):
        rows = slice(r * rt, (r + 1) * rt)
        run_min = jnp.full((rt, nl), jnp.inf, jnp.float32)
        run_j = jnp.zeros((rt, nl), jnp.int32)
        for j in range(k // nl):
            sj = dot[rows, j * nl:(j + 1) * nl] + csq[j * nl:(j + 1) * nl][None, :]
            mask = sj < run_min
            run_min = jnp.minimum(run_min, sj)
            run_j = jnp.where(mask, j, run_j)
        m = jnp.min(run_min, axis=1, keepdims=True)
        tie = run_min == m
        glob = run_j * nl + lane_iota
        idx_ref[rows] = jnp.min(jnp.where(tie, glob, k), axis=1)
        acc_ref[rows, :] += zsq[rows, :] + m

    @pl.when(i == nsteps - 1)
    def _():
        lsum_ref[0, 0] = jnp.sum(acc_ref[...])


def _dist_chunk(z_chunk, cb_wide, csq):
    nc, d = z_chunk.shape
    k = cb_wide.shape[0]
    nb = nc // _BN
    idx, lsum = pl.pallas_call(
        _dist_body,
        grid=(nb,),
        in_specs=[
            pl.BlockSpec((_BN, d), lambda i: (i, 0)),
            pl.BlockSpec((k, _WIDE), lambda i: (0, 0)),
            pl.BlockSpec((k,), lambda i: (0,)),
        ],
        out_specs=[
            pl.BlockSpec((_BN,), lambda i: (i,)),
            pl.BlockSpec(memory_space=pltpu.SMEM),
        ],
        out_shape=[
            jax.ShapeDtypeStruct((nc,), jnp.int32),
            jax.ShapeDtypeStruct((1, 1), jnp.float32),
        ],
        scratch_shapes=[
            pltpu.VMEM((_BN, 1), jnp.float32),
            pltpu.VMEM((_BN, _WIDE), jnp.float32),
        ],
        compiler_params=pltpu.CompilerParams(
            dimension_semantics=("arbitrary",)),
    )(z_chunk, cb_wide, csq)
    return idx, lsum


def _sc_gather_narrow(cb_wide, indices, d_out):
    n = indices.shape[0]
    dw = cb_wide.shape[1]
    idx2 = indices.reshape(1, n)
    mesh = plsc.VectorSubcoreMesh(
        core_axis_name="core", subcore_axis_name="subcore")

    @functools.partial(
        pl.kernel,
        out_type=jax.ShapeDtypeStruct((n, d_out), cb_wide.dtype),
        mesh=mesh,
        scratch_types=[pltpu.VMEM((_GATHER_W, dw), cb_wide.dtype)])
    def gather_kernel(cb_hbm, i_hbm, o_hbm, g_scratch):
        def body(i_vmem, o_vmem):
            pltpu.sync_copy(cb_hbm.at[i_vmem.at[0]], g_scratch)
            o_vmem[...] = g_scratch[:, :d_out]

        pltpu.emit_pipeline(
            body,
            grid=(n // _GATHER_W,),
            in_specs=[pl.BlockSpec((1, _GATHER_W), index_map=lambda i: (0, i))],
            out_specs=[pl.BlockSpec((_GATHER_W, d_out),
                                    index_map=lambda i: (i, 0))],
            core_axis_name=("core", "subcore"),
            dimension_semantics=(pltpu.PARALLEL,),
        )(i_hbm, o_hbm)

    return gather_kernel(cb_wide, idx2)


def kernel(z, codebook):
    n, d = z.shape
    cb_wide, csq = _prep(codebook)
    nc = n // _CHUNKS
    idxs, lsums, quants = [], [], []
    for c in range(_CHUNKS):
        zc = jax.lax.slice_in_dim(z, c * nc, (c + 1) * nc, axis=0)
        idx_c, lsum_c = _dist_chunk(zc, cb_wide, csq)
        idxs.append(idx_c)
        lsums.append(lsum_c)
        quants.append(_sc_gather_narrow(cb_wide, idx_c, d))
    idx = jnp.concatenate(idxs) if _CHUNKS > 1 else idxs[0]
    quantized = (jnp.concatenate(quants, axis=0)
                 if _CHUNKS > 1 else quants[0])
    loss = sum(ls[0, 0] for ls in lsums) / float(n * d)
    return (quantized, idx, loss)


# speedup vs baseline: 1.1659x; 1.0016x over previous
import functools

import jax
import jax.numpy as jnp
from jax.experimental import pallas as pl
from jax.experimental.pallas import tpu as pltpu
from jax.experimental.pallas import tpu_sc as plsc

_BN = 512
_GATHER_W = 256
_WIDE = 128
_CHUNKS = 2


def _prep_body(cb_ref, cbw_ref, csq_ref):
    d = cb_ref.shape[1]
    cb = cb_ref[...]
    cbw_ref[:, :d] = cb
    cbw_ref[:, d:] = jnp.zeros((cb.shape[0], _WIDE - d), jnp.float32)
    csq_ref[...] = jnp.sum(cb * cb, axis=1)


def _prep(codebook):
    k, _ = codebook.shape
    return pl.pallas_call(
        _prep_body,
        out_shape=[
            jax.ShapeDtypeStruct((k, _WIDE), jnp.float32),
            jax.ShapeDtypeStruct((k,), jnp.float32),
        ],
    )(codebook)


def _dist_body(z_ref, cbw_ref, csq_ref, idx_ref, lsum_ref, acc_ref, zp_ref):
    i = pl.program_id(0)
    nsteps = pl.num_programs(0)
    d = z_ref.shape[1]
    bn = z_ref.shape[0]

    @pl.when(i == 0)
    def _():
        acc_ref[...] = jnp.zeros_like(acc_ref)
        zp_ref[:, d:] = jnp.zeros((bn, _WIDE - d), jnp.float32)

    z = z_ref[...]
    zsq = jnp.sum(z * z, axis=1, keepdims=True)
    zp_ref[:, :d] = z * -2.0
    dot = jax.lax.dot_general(
        zp_ref[...], cbw_ref[...], (((1,), (1,)), ((), ())),
        preferred_element_type=jnp.float32)
    k = dot.shape[1]
    csq = csq_ref[...]
    nl = 128
    rt = 64
    lane_iota = jax.lax.broadcasted_iota(jnp.int32, (rt, nl), 1)
    for r in range(bn // rt):
        rows = slice(r * rt, (r + 1) * rt)
        run_min = jnp.full((rt, nl), jnp.inf, jnp.float32)
        run_j = jnp.zeros((rt, nl), jnp.int32)
        for j in range(k // nl):
            sj = dot[rows, j * nl:(j + 1) * nl] + csq[j * nl:(j + 1) * nl][None, :]
            mask = sj < run_min
            run_min = jnp.minimum(run_min, sj)
            run_j = jnp.where(mask, j, run_j)
        m = jnp.min(run_min, axis=1, keepdims=True)
        tie = run_min == m
        glob = run_j * nl + lane_iota
        idx_ref[rows] = jnp.min(jnp.where(tie, glob, k), axis=1)
        acc_ref[rows, :] += zsq[rows, :] + m

    @pl.when(i == nsteps - 1)
    def _():
        lsum_ref[0, 0] = jnp.sum(acc_ref[...])


def _dist_chunk(z_chunk, cb_wide, csq):
    nc, d = z_chunk.shape
    k = cb_wide.shape[0]
    nb = nc // _BN
    idx, lsum = pl.pallas_call(
        _dist_body,
        grid=(nb,),
        in_specs=[
            pl.BlockSpec((_BN, d), lambda i: (i, 0)),
            pl.BlockSpec((k, _WIDE), lambda i: (0, 0)),
            pl.BlockSpec((k,), lambda i: (0,)),
        ],
        out_specs=[
            pl.BlockSpec((_BN,), lambda i: (i,)),
            pl.BlockSpec(memory_space=pltpu.SMEM),
        ],
        out_shape=[
            jax.ShapeDtypeStruct((nc,), jnp.int32),
            jax.ShapeDtypeStruct((1, 1), jnp.float32),
        ],
        scratch_shapes=[
            pltpu.VMEM((_BN, 1), jnp.float32),
            pltpu.VMEM((_BN, _WIDE), jnp.float32),
        ],
        compiler_params=pltpu.CompilerParams(
            dimension_semantics=("arbitrary",)),
    )(z_chunk, cb_wide, csq)
    return idx, lsum


def _sc_gather_narrow(cb_wide, indices, d_out):
    n = indices.shape[0]
    dw = cb_wide.shape[1]
    idx2 = indices.reshape(1, n)
    mesh = plsc.VectorSubcoreMesh(
        core_axis_name="core", subcore_axis_name="subcore")

    @functools.partial(
        pl.kernel,
        out_type=jax.ShapeDtypeStruct((n, d_out), cb_wide.dtype),
        mesh=mesh,
        scratch_types=[pltpu.VMEM((_GATHER_W, dw), cb_wide.dtype)])
    def gather_kernel(cb_hbm, i_hbm, o_hbm, g_scratch):
        def body(i_vmem, o_vmem):
            pltpu.sync_copy(cb_hbm.at[i_vmem.at[0]], g_scratch)
            o_vmem[...] = g_scratch[:, :d_out]

        pltpu.emit_pipeline(
            body,
            grid=(n // _GATHER_W,),
            in_specs=[pl.BlockSpec((1, _GATHER_W), index_map=lambda i: (0, i))],
            out_specs=[pl.BlockSpec((_GATHER_W, d_out),
                                    index_map=lambda i: (i, 0))],
            core_axis_name=("core", "subcore"),
            dimension_semantics=(pltpu.PARALLEL,),
        )(i_hbm, o_hbm)

    return gather_kernel(cb_wide, idx2)


def kernel(z, codebook):
    n, d = z.shape
    cb_wide, csq = _prep(codebook)
    nc = n // _CHUNKS
    idxs, lsums, quants = [], [], []
    for c in range(_CHUNKS):
        zc = jax.lax.slice_in_dim(z, c * nc, (c + 1) * nc, axis=0)
        idx_c, lsum_c = _dist_chunk(zc, cb_wide, csq)
        idxs.append(idx_c)
        lsums.append(lsum_c)
        quants.append(_sc_gather_narrow(cb_wide, idx_c, d))
    idx = jnp.concatenate(idxs) if _CHUNKS > 1 else idxs[0]
    quantized = (jnp.concatenate(quants, axis=0)
                 if _CHUNKS > 1 else quants[0])
    loss = sum(ls[0, 0] for ls in lsums) / float(n * d)
    return (quantized, idx, loss)


# speedup vs baseline: 1.1921x; 1.0225x over previous
import functools

import jax
import jax.numpy as jnp
from jax.experimental import pallas as pl
from jax.experimental.pallas import tpu as pltpu
from jax.experimental.pallas import tpu_sc as plsc

_BN = 512
_GATHER_W = 256
_WIDE = 128
_CHUNKS = 2
_KSPLIT = 4


def _prep_body(cb_ref, cbw_ref, csq_ref):
    d = cb_ref.shape[1]
    cb = cb_ref[...]
    cbw_ref[:, :d] = cb
    cbw_ref[:, d:] = jnp.zeros((cb.shape[0], _WIDE - d), jnp.float32)
    csq_ref[...] = jnp.sum(cb * cb, axis=1)


def _prep(codebook):
    k, _ = codebook.shape
    return pl.pallas_call(
        _prep_body,
        out_shape=[
            jax.ShapeDtypeStruct((k, _WIDE), jnp.float32),
            jax.ShapeDtypeStruct((k,), jnp.float32),
        ],
    )(codebook)


def _dist_body(z_ref, cbw_ref, csq_ref, idx_ref, lsum_ref, acc_ref, zp_ref,
               rmin_ref, rj_ref):
    i = pl.program_id(0)
    nsteps = pl.num_programs(0)
    d = z_ref.shape[1]
    bn = z_ref.shape[0]
    k = cbw_ref.shape[0]
    nl = 128
    rt = 64

    @pl.when(i == 0)
    def _():
        acc_ref[...] = jnp.zeros_like(acc_ref)
        zp_ref[:, d:] = jnp.zeros((bn, _WIDE - d), jnp.float32)
        rmin_ref[...] = jnp.zeros_like(rmin_ref)
        rj_ref[...] = jnp.zeros_like(rj_ref)

    f_extract = jnp.where(i > 0, 1.0, 0.0)
    f_compute = jnp.where(i < nsteps - 1, 1.0, 0.0)

    lane_iota = jax.lax.broadcasted_iota(jnp.int32, (rt, nl), 1)
    for r in range(bn // rt):
        rows = slice(r * rt, (r + 1) * rt)
        run_min = rmin_ref[rows, :]
        run_j = rj_ref[rows, :]
        m = jnp.min(run_min, axis=1, keepdims=True)
        tie = run_min == m
        glob = run_j * nl + lane_iota
        idx_ref[rows] = jnp.min(jnp.where(tie, glob, k), axis=1)
        acc_ref[rows, :] += m * f_extract

    z = z_ref[...]
    zsq = jnp.sum(z * z, axis=1, keepdims=True)
    zp_ref[:, :d] = z * -2.0
    dot = jax.lax.dot_general(
        zp_ref[...], cbw_ref[...], (((1,), (1,)), ((), ())),
        preferred_element_type=jnp.float32)
    csq = csq_ref[...]
    for r in range(bn // rt):
        rows = slice(r * rt, (r + 1) * rt)
        run_min = jnp.full((rt, nl), jnp.inf, jnp.float32)
        run_j = jnp.zeros((rt, nl), jnp.int32)
        for j in range(k // nl):
            sj = dot[rows, j * nl:(j + 1) * nl] \
                + csq[j * nl:(j + 1) * nl][None, :]
            mask = sj < run_min
            run_min = jnp.minimum(run_min, sj)
            run_j = jnp.where(mask, j, run_j)
        rmin_ref[rows, :] = run_min
        rj_ref[rows, :] = run_j
    acc_ref[...] += zsq * f_compute

    @pl.when(i == nsteps - 1)
    def _():
        lsum_ref[0, 0] = jnp.sum(acc_ref[...])


def _dist_chunk(z_chunk, cb_wide, csq):
    nc, d = z_chunk.shape
    k = cb_wide.shape[0]
    nb = nc // _BN
    idx, lsum = pl.pallas_call(
        _dist_body,
        grid=(nb + 1,),
        in_specs=[
            pl.BlockSpec((_BN, d), lambda i: (jnp.minimum(i, nb - 1), 0)),
            pl.BlockSpec((k, _WIDE), lambda i: (0, 0)),
            pl.BlockSpec((k,), lambda i: (0,)),
        ],
        out_specs=[
            pl.BlockSpec((_BN,), lambda i: (jnp.maximum(i - 1, 0),)),
            pl.BlockSpec(memory_space=pltpu.SMEM),
        ],
        out_shape=[
            jax.ShapeDtypeStruct((nc,), jnp.int32),
            jax.ShapeDtypeStruct((1, 1), jnp.float32),
        ],
        scratch_shapes=[
            pltpu.VMEM((_BN, 1), jnp.float32),
            pltpu.VMEM((_BN, _WIDE), jnp.float32),
            pltpu.VMEM((_BN, 128), jnp.float32),
            pltpu.VMEM((_BN, 128), jnp.int32),
        ],
        compiler_params=pltpu.CompilerParams(
            dimension_semantics=("arbitrary",)),
    )(z_chunk, cb_wide, csq)
    return idx, lsum


def _sc_gather_narrow(cb_wide, indices, d_out):
    n = indices.shape[0]
    dw = cb_wide.shape[1]
    idx2 = indices.reshape(1, n)
    mesh = plsc.VectorSubcoreMesh(
        core_axis_name="core", subcore_axis_name="subcore")

    @functools.partial(
        pl.kernel,
        out_type=jax.ShapeDtypeStruct((n, d_out), cb_wide.dtype),
        mesh=mesh,
        scratch_types=[pltpu.VMEM((_GATHER_W, dw), cb_wide.dtype)])
    def gather_kernel(cb_hbm, i_hbm, o_hbm, g_scratch):
        def body(i_vmem, o_vmem):
            pltpu.sync_copy(cb_hbm.at[i_vmem.at[0]], g_scratch)
            o_vmem[...] = g_scratch[:, :d_out]

        pltpu.emit_pipeline(
            body,
            grid=(n // _GATHER_W,),
            in_specs=[pl.BlockSpec((1, _GATHER_W), index_map=lambda i: (0, i))],
            out_specs=[pl.BlockSpec((_GATHER_W, d_out),
                                    index_map=lambda i: (i, 0))],
            core_axis_name=("core", "subcore"),
            dimension_semantics=(pltpu.PARALLEL,),
        )(i_hbm, o_hbm)

    return gather_kernel(cb_wide, idx2)


def kernel(z, codebook):
    n, d = z.shape
    cb_wide, csq = _prep(codebook)
    nc = n // _CHUNKS
    idxs, lsums, quants = [], [], []
    for c in range(_CHUNKS):
        zc = jax.lax.slice_in_dim(z, c * nc, (c + 1) * nc, axis=0)
        idx_c, lsum_c = _dist_chunk(zc, cb_wide, csq)
        idxs.append(idx_c)
        lsums.append(lsum_c)
        quants.append(_sc_gather_narrow(cb_wide, idx_c, d))
    idx = jnp.concatenate(idxs) if _CHUNKS > 1 else idxs[0]
    quantized = (jnp.concatenate(quants, axis=0)
                 if _CHUNKS > 1 else quants[0])
    loss = sum(ls[0, 0] for ls in lsums) / float(n * d)
    return (quantized, idx, loss)


# speedup vs baseline: 1.1989x; 1.0057x over previous
import functools

import jax
import jax.numpy as jnp
from jax.experimental import pallas as pl
from jax.experimental.pallas import tpu as pltpu
from jax.experimental.pallas import tpu_sc as plsc

_BN = 1024
_GATHER_W = 256
_WIDE = 128
_CHUNKS = 2
_KSPLIT = 4


def _prep_body(cb_ref, cbw_ref, csq_ref):
    d = cb_ref.shape[1]
    cb = cb_ref[...]
    cbw_ref[:, :d] = cb
    cbw_ref[:, d:] = jnp.zeros((cb.shape[0], _WIDE - d), jnp.float32)
    csq_ref[...] = jnp.sum(cb * cb, axis=1)


def _prep(codebook):
    k, _ = codebook.shape
    return pl.pallas_call(
        _prep_body,
        out_shape=[
            jax.ShapeDtypeStruct((k, _WIDE), jnp.float32),
            jax.ShapeDtypeStruct((k,), jnp.float32),
        ],
    )(codebook)


def _dist_body(z_ref, cbw_ref, csq_ref, idx_ref, lsum_ref, acc_ref, zp_ref,
               rmin_ref, rj_ref):
    i = pl.program_id(0)
    nsteps = pl.num_programs(0)
    d = z_ref.shape[1]
    bn = z_ref.shape[0]
    k = cbw_ref.shape[0]
    nl = 128
    rt = 64

    @pl.when(i == 0)
    def _():
        acc_ref[...] = jnp.zeros_like(acc_ref)
        zp_ref[:, d:] = jnp.zeros((bn, _WIDE - d), jnp.float32)
        rmin_ref[...] = jnp.zeros_like(rmin_ref)
        rj_ref[...] = jnp.zeros_like(rj_ref)

    f_extract = jnp.where(i > 0, 1.0, 0.0)
    f_compute = jnp.where(i < nsteps - 1, 1.0, 0.0)

    lane_iota = jax.lax.broadcasted_iota(jnp.int32, (rt, nl), 1)
    for r in range(bn // rt):
        rows = slice(r * rt, (r + 1) * rt)
        run_min = rmin_ref[rows, :]
        run_j = rj_ref[rows, :]
        m = jnp.min(run_min, axis=1, keepdims=True)
        tie = run_min == m
        glob = run_j * nl + lane_iota
        idx_ref[rows] = jnp.min(jnp.where(tie, glob, k), axis=1)
        acc_ref[rows, :] += m * f_extract

    z = z_ref[...]
    zsq = jnp.sum(z * z, axis=1, keepdims=True)
    zp_ref[:, :d] = z * -2.0
    dot = jax.lax.dot_general(
        zp_ref[...], cbw_ref[...], (((1,), (1,)), ((), ())),
        preferred_element_type=jnp.float32)
    csq = csq_ref[...]
    for r in range(bn // rt):
        rows = slice(r * rt, (r + 1) * rt)
        run_min = jnp.full((rt, nl), jnp.inf, jnp.float32)
        run_j = jnp.zeros((rt, nl), jnp.int32)
        for j in range(k // nl):
            sj = dot[rows, j * nl:(j + 1) * nl] \
                + csq[j * nl:(j + 1) * nl][None, :]
            mask = sj < run_min
            run_min = jnp.minimum(run_min, sj)
            run_j = jnp.where(mask, j, run_j)
        rmin_ref[rows, :] = run_min
        rj_ref[rows, :] = run_j
    acc_ref[...] += zsq * f_compute

    @pl.when(i == nsteps - 1)
    def _():
        lsum_ref[0, 0] = jnp.sum(acc_ref[...])


def _dist_chunk(z_chunk, cb_wide, csq):
    nc, d = z_chunk.shape
    k = cb_wide.shape[0]
    nb = nc // _BN
    idx, lsum = pl.pallas_call(
        _dist_body,
        grid=(nb + 1,),
        in_specs=[
            pl.BlockSpec((_BN, d), lambda i: (jnp.minimum(i, nb - 1), 0)),
            pl.BlockSpec((k, _WIDE), lambda i: (0, 0)),
            pl.BlockSpec((k,), lambda i: (0,)),
        ],
        out_specs=[
            pl.BlockSpec((_BN,), lambda i: (jnp.maximum(i - 1, 0),)),
            pl.BlockSpec(memory_space=pltpu.SMEM),
        ],
        out_shape=[
            jax.ShapeDtypeStruct((nc,), jnp.int32),
            jax.ShapeDtypeStruct((1, 1), jnp.float32),
        ],
        scratch_shapes=[
            pltpu.VMEM((_BN, 1), jnp.float32),
            pltpu.VMEM((_BN, _WIDE), jnp.float32),
            pltpu.VMEM((_BN, 128), jnp.float32),
            pltpu.VMEM((_BN, 128), jnp.int32),
        ],
        compiler_params=pltpu.CompilerParams(
            dimension_semantics=("arbitrary",)),
    )(z_chunk, cb_wide, csq)
    return idx, lsum


def _sc_gather_narrow(cb_wide, indices, d_out):
    n = indices.shape[0]
    dw = cb_wide.shape[1]
    idx2 = indices.reshape(1, n)
    mesh = plsc.VectorSubcoreMesh(
        core_axis_name="core", subcore_axis_name="subcore")

    @functools.partial(
        pl.kernel,
        out_type=jax.ShapeDtypeStruct((n, d_out), cb_wide.dtype),
        mesh=mesh,
        scratch_types=[pltpu.VMEM((_GATHER_W, dw), cb_wide.dtype)])
    def gather_kernel(cb_hbm, i_hbm, o_hbm, g_scratch):
        def body(i_vmem, o_vmem):
            pltpu.sync_copy(cb_hbm.at[i_vmem.at[0]], g_scratch)
            o_vmem[...] = g_scratch[:, :d_out]

        pltpu.emit_pipeline(
            body,
            grid=(n // _GATHER_W,),
            in_specs=[pl.BlockSpec((1, _GATHER_W), index_map=lambda i: (0, i))],
            out_specs=[pl.BlockSpec((_GATHER_W, d_out),
                                    index_map=lambda i: (i, 0))],
            core_axis_name=("core", "subcore"),
            dimension_semantics=(pltpu.PARALLEL,),
        )(i_hbm, o_hbm)

    return gather_kernel(cb_wide, idx2)


def kernel(z, codebook):
    n, d = z.shape
    cb_wide, csq = _prep(codebook)
    nc = n // _CHUNKS
    idxs, lsums, quants = [], [], []
    for c in range(_CHUNKS):
        zc = jax.lax.slice_in_dim(z, c * nc, (c + 1) * nc, axis=0)
        idx_c, lsum_c = _dist_chunk(zc, cb_wide, csq)
        idxs.append(idx_c)
        lsums.append(lsum_c)
        quants.append(_sc_gather_narrow(cb_wide, idx_c, d))
    idx = jnp.concatenate(idxs) if _CHUNKS > 1 else idxs[0]
    quantized = (jnp.concatenate(quants, axis=0)
                 if _CHUNKS > 1 else quants[0])
    loss = sum(ls[0, 0] for ls in lsums) / float(n * d)
    return (quantized, idx, loss)


# speedup vs baseline: 1.2013x; 1.0020x over previous
import functools

import jax
import jax.numpy as jnp
from jax.experimental import pallas as pl
from jax.experimental.pallas import tpu as pltpu
from jax.experimental.pallas import tpu_sc as plsc

_BN = 1024
_GATHER_W = 256
_WIDE = 128
_CHUNKS = 2


def _prep_body(cb_ref, cbw_ref, csq_ref):
    d = cb_ref.shape[1]
    cb = cb_ref[...]
    cbw_ref[:, :d] = cb
    cbw_ref[:, d:] = jnp.zeros((cb.shape[0], _WIDE - d), jnp.float32)
    csq_ref[...] = jnp.sum(cb * cb, axis=1)


def _prep(codebook):
    k, _ = codebook.shape
    return pl.pallas_call(
        _prep_body,
        out_shape=[
            jax.ShapeDtypeStruct((k, _WIDE), jnp.float32),
            jax.ShapeDtypeStruct((k,), jnp.float32),
        ],
    )(codebook)


def _dist_body(z_ref, cbw_ref, csq_ref, idx_ref, lsum_ref, acc_ref, zp_ref,
               rmin_ref, rj_ref):
    i = pl.program_id(0)
    nsteps = pl.num_programs(0)
    d = z_ref.shape[1]
    bn = z_ref.shape[0]
    k = cbw_ref.shape[0]
    nl = 128
    rt = 64

    @pl.when(i == 0)
    def _():
        acc_ref[...] = jnp.zeros_like(acc_ref)
        zp_ref[:, d:] = jnp.zeros((bn, _WIDE - d), jnp.float32)
        rmin_ref[...] = jnp.zeros_like(rmin_ref)
        rj_ref[...] = jnp.zeros_like(rj_ref)

    f_extract = jnp.where(i > 0, 1.0, 0.0)
    f_compute = jnp.where(i < nsteps - 1, 1.0, 0.0)

    lane_iota = jax.lax.broadcasted_iota(jnp.int32, (rt, nl), 1)
    for r in range(bn // rt):
        rows = slice(r * rt, (r + 1) * rt)
        run_min = rmin_ref[rows, :]
        run_j = rj_ref[rows, :]
        m = jnp.min(run_min, axis=1, keepdims=True)
        tie = run_min == m
        glob = run_j * nl + lane_iota
        idx_ref[rows] = jnp.min(jnp.where(tie, glob, k), axis=1)
        acc_ref[rows, :] += m * f_extract

    z = z_ref[...]
    zsq = jnp.sum(z * z, axis=1, keepdims=True)
    zp_ref[:, :d] = z * -2.0
    dot = jax.lax.dot_general(
        zp_ref[...], cbw_ref[...], (((1,), (1,)), ((), ())),
        preferred_element_type=jnp.float32)
    csq = csq_ref[...]
    for r in range(bn // rt):
        rows = slice(r * rt, (r + 1) * rt)
        run_min = jnp.full((rt, nl), jnp.inf, jnp.float32)
        run_j = jnp.zeros((rt, nl), jnp.int32)
        for j in range(k // nl):
            sj = dot[rows, j * nl:(j + 1) * nl] \
                + csq[j * nl:(j + 1) * nl][None, :]
            mask = sj < run_min
            run_min = jnp.minimum(run_min, sj)
            run_j = jnp.where(mask, j, run_j)
        rmin_ref[rows, :] = run_min
        rj_ref[rows, :] = run_j
    acc_ref[...] += zsq * f_compute

    @pl.when(i == nsteps - 1)
    def _():
        lsum_ref[0, 0] = jnp.sum(acc_ref[...])


def _dist_chunk(z_chunk, cb_wide, csq):
    nc, d = z_chunk.shape
    k = cb_wide.shape[0]
    nb = nc // _BN
    idx, lsum = pl.pallas_call(
        _dist_body,
        grid=(nb + 1,),
        in_specs=[
            pl.BlockSpec((_BN, d), lambda i: (jnp.minimum(i, nb - 1), 0)),
            pl.BlockSpec((k, _WIDE), lambda i: (0, 0)),
            pl.BlockSpec((k,), lambda i: (0,)),
        ],
        out_specs=[
            pl.BlockSpec((_BN,), lambda i: (jnp.maximum(i - 1, 0),)),
            pl.BlockSpec(memory_space=pltpu.SMEM),
        ],
        out_shape=[
            jax.ShapeDtypeStruct((nc,), jnp.int32),
            jax.ShapeDtypeStruct((1, 1), jnp.float32),
        ],
        scratch_shapes=[
            pltpu.VMEM((_BN, 1), jnp.float32),
            pltpu.VMEM((_BN, _WIDE), jnp.float32),
            pltpu.VMEM((_BN, 128), jnp.float32),
            pltpu.VMEM((_BN, 128), jnp.int32),
        ],
        compiler_params=pltpu.CompilerParams(
            dimension_semantics=("arbitrary",)),
    )(z_chunk, cb_wide, csq)
    return idx, lsum


def _sc_gather_narrow(cb_wide, indices, d_out):
    n = indices.shape[0]
    dw = cb_wide.shape[1]
    idx2 = indices.reshape(1, n)
    mesh = plsc.VectorSubcoreMesh(
        core_axis_name="core", subcore_axis_name="subcore")

    @functools.partial(
        pl.kernel,
        out_type=jax.ShapeDtypeStruct((n, d_out), cb_wide.dtype),
        mesh=mesh,
        scratch_types=[pltpu.VMEM((_GATHER_W, dw), cb_wide.dtype)])
    def gather_kernel(cb_hbm, i_hbm, o_hbm, g_scratch):
        def body(i_vmem, o_vmem):
            pltpu.sync_copy(cb_hbm.at[i_vmem.at[0]], g_scratch)
            o_vmem[...] = g_scratch[:, :d_out]

        pltpu.emit_pipeline(
            body,
            grid=(n // _GATHER_W,),
            in_specs=[pl.BlockSpec((1, _GATHER_W), index_map=lambda i: (0, i))],
            out_specs=[pl.BlockSpec((_GATHER_W, d_out),
                                    index_map=lambda i: (i, 0))],
            core_axis_name=("core", "subcore"),
            dimension_semantics=(pltpu.PARALLEL,),
        )(i_hbm, o_hbm)

    return gather_kernel(cb_wide, idx2)


def kernel(z, codebook):
    n, d = z.shape
    cb_wide, csq = _prep(codebook)
    nc = n // _CHUNKS
    idxs, lsums, quants = [], [], []
    for c in range(_CHUNKS):
        zc = jax.lax.slice_in_dim(z, c * nc, (c + 1) * nc, axis=0)
        idx_c, lsum_c = _dist_chunk(zc, cb_wide, csq)
        idxs.append(idx_c)
        lsums.append(lsum_c)
        quants.append(_sc_gather_narrow(cb_wide, idx_c, d))
    idx = jnp.concatenate(idxs) if _CHUNKS > 1 else idxs[0]
    quantized = (jnp.concatenate(quants, axis=0)
                 if _CHUNKS > 1 else quants[0])
    loss = sum(ls[0, 0] for ls in lsums) / float(n * d)
    return (quantized, idx, loss)
